# baseline (speedup 1.0000x reference)
import functools

import jax
import jax.numpy as jnp
from jax import lax
from jax.experimental import pallas as pl
from jax.experimental.pallas import tpu as pltpu
from jax.experimental.pallas import tpu_sc as plsc

_T = 16384
_D = 2048
_E = 64
_K = 2
_BT = 2048

_NC = 2
_NS = 16
_L = 16
_NW = _NC * _NS
_TW = _T // _NW


def _gate_body(x_ref, w_ref, probst_ref):
    logits = jax.lax.dot_general(
        w_ref[...], x_ref[...], (((0,), (1,)), ((), ())),
        preferred_element_type=jnp.float32)
    m1 = jnp.max(logits, axis=0, keepdims=True)
    e = jnp.exp(logits - m1)
    s = jnp.sum(e, axis=0, keepdims=True)
    probst_ref[...] = e * (1.0 / s)


@functools.partial(
    pl.kernel,
    mesh=plsc.VectorSubcoreMesh(core_axis_name="c", subcore_axis_name="s"),
    out_type=jax.ShapeDtypeStruct((8, _T), jnp.float32),
    scratch_types=[
        pltpu.VMEM((_E, _TW), jnp.float32),
        pltpu.VMEM((4, _TW), jnp.float32),
    ],
)
def _sc_top2(probst_hbm, aux_hbm, pbuf, obuf):
    wid = lax.axis_index("s") * _NC + lax.axis_index("c")
    base = wid * _TW
    pltpu.sync_copy(probst_hbm.at[:, pl.ds(base, _TW)], pbuf)

    def group(g, carry):
        off = g * _L
        m1 = pbuf[0, pl.ds(off, _L)]
        i1 = jnp.zeros((_L,), jnp.float32)
        m2 = jnp.full((_L,), -1.0, jnp.float32)
        i2 = jnp.zeros((_L,), jnp.float32)
        for e in range(1, _E):
            p = pbuf[e, pl.ds(off, _L)]
            ef = jnp.full((_L,), float(e), jnp.float32)
            gt1 = p > m1
            gt2 = p > m2
            m2 = jnp.where(gt1, m1, jnp.where(gt2, p, m2))
            i2 = jnp.where(gt1, i1, jnp.where(gt2, ef, i2))
            m1 = jnp.where(gt1, p, m1)
            i1 = jnp.where(gt1, ef, i1)
        rd = 1.0 / (m1 + m2 + 1e-9)
        obuf[0, pl.ds(off, _L)] = m1 * rd
        obuf[1, pl.ds(off, _L)] = m2 * rd
        obuf[2, pl.ds(off, _L)] = i1
        obuf[3, pl.ds(off, _L)] = i2
        return carry

    lax.fori_loop(0, _TW // _L, group, 0)
    pltpu.sync_copy(obuf, aux_hbm.at[pl.ds(0, 4), pl.ds(base, _TW)])


@jax.jit
def kernel(x, W_gate):
    probst = pl.pallas_call(
        _gate_body,
        grid=(_T // _BT,),
        in_specs=[
            pl.BlockSpec((_BT, _D), lambda i: (i, 0)),
            pl.BlockSpec((_D, _E), lambda i: (0, 0)),
        ],
        out_specs=pl.BlockSpec((_E, _BT), lambda i: (0, i)),
        out_shape=jax.ShapeDtypeStruct((_E, _T), jnp.float32),
        compiler_params=pltpu.CompilerParams(
            dimension_semantics=("arbitrary",),
        ),
    )(x, W_gate)
    aux = _sc_top2(probst)
    tkp = aux[0:2].T
    tki = aux[2:4].T.astype(jnp.int32)
    return (tkp, tki, probst.T)


# speedup vs baseline: 1.4406x; 1.4406x over previous
import jax
import jax.numpy as jnp
from jax.experimental import pallas as pl
from jax.experimental.pallas import tpu as pltpu

_T = 16384
_D = 2048
_E = 64
_K = 2
_BT = 2048


def _router_body(x_ref, w_ref, aux_ref, probst_ref):
    logits = jax.lax.dot_general(
        w_ref[...], x_ref[...], (((0,), (1,)), ((), ())),
        preferred_element_type=jnp.float32)

    m1 = jnp.max(logits, axis=0, keepdims=True)
    e = jnp.exp(logits - m1)
    s = jnp.sum(e, axis=0, keepdims=True)
    probst_ref[...] = e * (1.0 / s)

    iota = jax.lax.broadcasted_iota(jnp.int32, logits.shape, 0).astype(jnp.float32)
    i1 = jnp.min(jnp.where(logits == m1, iota, float(_E)), axis=0, keepdims=True)
    masked = jnp.where(iota == i1, -jnp.inf, logits)
    l2 = jnp.max(masked, axis=0, keepdims=True)
    i2 = jnp.min(jnp.where(masked == l2, iota, float(_E)), axis=0, keepdims=True)

    e2 = jnp.exp(l2 - m1)
    t1 = 1.0 / (1.0 + e2 + 1e-9 * s)
    aux_ref[0:4, :] = jnp.concatenate([t1, e2 * t1, i1, i2], axis=0)


@jax.jit
def kernel(x, W_gate):
    aux, probst = pl.pallas_call(
        _router_body,
        grid=(_T // _BT,),
        in_specs=[
            pl.BlockSpec((_BT, _D), lambda i: (i, 0)),
            pl.BlockSpec((_D, _E), lambda i: (0, 0)),
        ],
        out_specs=[
            pl.BlockSpec((8, _BT), lambda i: (0, i)),
            pl.BlockSpec((_E, _BT), lambda i: (0, i)),
        ],
        out_shape=[
            jax.ShapeDtypeStruct((8, _T), jnp.float32),
            jax.ShapeDtypeStruct((_E, _T), jnp.float32),
        ],
        compiler_params=pltpu.CompilerParams(
            dimension_semantics=("arbitrary",),
        ),
    )(x, W_gate)
    tkp = aux[0:2].T
    tki = aux[2:4].T.astype(jnp.int32)
    return (tkp, tki, probst.T)
